# baseline (speedup 1.0000x reference)
import functools

import jax
import jax.numpy as jnp
from jax import lax
from jax.experimental import pallas as pl
from jax.experimental.pallas import tpu as pltpu
from jax.experimental.pallas import tpu_sc as plsc

N_NODES = 100000
N_GRAPHS = 256
D_FEAT = 128

NC = 2
NS = 16
NW = NC * NS
LANES = 16
NJ = D_FEAT // LANES

CHUNK = 160
NCHUNKS = N_NODES // CHUNK
NSLOT = 4
T_MAX = (NCHUNKS + NW - 1) // NW

_MESH = plsc.VectorSubcoreMesh(core_axis_name="c", subcore_axis_name="s")


@functools.partial(
    pl.kernel,
    out_type=jax.ShapeDtypeStruct((N_NODES, D_FEAT), jnp.float32),
    mesh=_MESH,
    scratch_types=[
        pltpu.VMEM((N_GRAPHS, D_FEAT), jnp.float32),
        pltpu.VMEM((N_GRAPHS,), jnp.float32),
        pltpu.VMEM((D_FEAT,), jnp.float32),
        pltpu.VMEM((CHUNK, D_FEAT), jnp.float32),
        pltpu.VMEM((CHUNK, D_FEAT), jnp.float32),
        pltpu.VMEM((CHUNK, D_FEAT), jnp.float32),
        pltpu.VMEM((CHUNK, D_FEAT), jnp.float32),
        pltpu.VMEM((CHUNK,), jnp.int32),
        pltpu.VMEM((CHUNK,), jnp.int32),
        pltpu.VMEM((CHUNK,), jnp.int32),
        pltpu.VMEM((CHUNK,), jnp.int32),
        pltpu.SemaphoreType.DMA,
        pltpu.SemaphoreType.DMA,
        pltpu.SemaphoreType.DMA,
        pltpu.SemaphoreType.DMA,
        pltpu.SemaphoreType.DMA,
        pltpu.SemaphoreType.DMA,
        pltpu.SemaphoreType.DMA,
        pltpu.SemaphoreType.DMA,
    ],
)
def _sc_gather_add(nf_hbm, tc_hbm, w_hbm, idx_hbm, out_hbm,
                   emb_v, tc_v, w_v, nf0, nf1, nf2, nf3, ix0, ix1, ix2, ix3,
                   in0, in1, in2, in3, out0, out1, out2, out3):
    nf_bufs = (nf0, nf1, nf2, nf3)
    idx_bufs = (ix0, ix1, ix2, ix3)
    in_sems = (in0, in1, in2, in3)
    out_sems = (out0, out1, out2, out3)
    wid = lax.axis_index("s") * NC + lax.axis_index("c")

    def chunk_of(t):
        return wid + t * NW

    def fire_in(t, b):
        base = chunk_of(t) * CHUNK
        pltpu.async_copy(idx_hbm.at[pl.ds(base, CHUNK)], idx_bufs[b],
                         in_sems[b])
        pltpu.async_copy(nf_hbm.at[pl.ds(base, CHUNK)], nf_bufs[b],
                         in_sems[b])

    def wait_in(t, b):
        base = chunk_of(t) * CHUNK
        pltpu.make_async_copy(idx_hbm.at[pl.ds(base, CHUNK)], idx_bufs[b],
                              in_sems[b]).wait()
        pltpu.make_async_copy(nf_hbm.at[pl.ds(base, CHUNK)], nf_bufs[b],
                              in_sems[b]).wait()

    def fire_out(t, b):
        base = chunk_of(t) * CHUNK
        pltpu.async_copy(nf_bufs[b], out_hbm.at[pl.ds(base, CHUNK)],
                         out_sems[b])

    def wait_out(t, b):
        base = chunk_of(t) * CHUNK
        pltpu.make_async_copy(nf_bufs[b], out_hbm.at[pl.ds(base, CHUNK)],
                              out_sems[b]).wait()

    def compute(b):
        nf = nf_bufs[b]
        ix = idx_bufs[b]
        first = ix[pl.ds(0, LANES)][0]
        last = ix[pl.ds(CHUNK - LANES, LANES)][LANES - 1]

        @pl.when(first == last)
        def _uniform():
            rows = [emb_v[first, pl.ds(j * LANES, LANES)] for j in range(NJ)]

            @pl.loop(0, CHUNK, step=LANES, unroll=2)
            def _grp(i0):
                for lane in range(LANES):
                    for j in range(NJ):
                        sl = pl.ds(j * LANES, LANES)
                        plsc.addupdate(nf.at[i0 + lane, sl], rows[j])

        @pl.when(first != last)
        def _mixed():
            @pl.loop(0, CHUNK, step=LANES)
            def _grp(i0):
                bvec = ix[pl.ds(i0, LANES)]
                gf = bvec[0]
                gl = bvec[LANES - 1]

                @pl.when(gf == gl)
                def _grp_uniform():
                    grows = [emb_v[gf, pl.ds(j * LANES, LANES)]
                             for j in range(NJ)]
                    for lane in range(LANES):
                        for j in range(NJ):
                            sl = pl.ds(j * LANES, LANES)
                            plsc.addupdate(nf.at[i0 + lane, sl], grows[j])

                @pl.when(gf != gl)
                def _grp_mixed():
                    for lane in range(LANES):
                        g = bvec[lane]
                        for j in range(NJ):
                            sl = pl.ds(j * LANES, LANES)
                            plsc.addupdate(nf.at[i0 + lane, sl], emb_v[g, sl])

    for t0 in range(2):
        @pl.when(chunk_of(t0) < NCHUNKS)
        def _(t0=t0):
            fire_in(t0, t0 % NSLOT)

    pltpu.sync_copy(tc_hbm, tc_v)
    pltpu.sync_copy(w_hbm, w_v)
    wrow = [w_v[pl.ds(j * LANES, LANES)] for j in range(NJ)]

    @pl.loop(0, N_GRAPHS, step=LANES)
    def _build(g0):
        tvec = tc_v[pl.ds(g0, LANES)]
        for lane in range(LANES):
            s = tvec[lane]
            for j in range(NJ):
                emb_v[g0 + lane, pl.ds(j * LANES, LANES)] = wrow[j] * s

    @pl.loop(0, T_MAX, step=NSLOT)
    def _body(tt):
        for k in range(NSLOT):
            t = tt + k
            ci = chunk_of(t)

            @pl.when(ci < NCHUNKS)
            def _(t=t, b=k, ci=ci):
                wait_in(t, b)
                compute(b)
                fire_out(t, b)
                b2 = (b + 2) % NSLOT

                @pl.when(jnp.logical_and(t >= 2, chunk_of(t + 2) < NCHUNKS))
                def _():
                    wait_out(t - 2, b2)

                @pl.when(chunk_of(t + 2) < NCHUNKS)
                def _():
                    fire_in(t + 2, b2)

    nchunks_mine = (NCHUNKS - wid + NW - 1) // NW

    for k in range(NSLOT):
        t_last = nchunks_mine - 1 - k

        @pl.when(t_last >= 0)
        def _(t_last=t_last):
            for b in range(NSLOT):
                @pl.when(t_last % NSLOT == b)
                def _(b=b):
                    wait_out(t_last, b)


def kernel(node_features, total_charge, batch, W):
    idx = batch.astype(jnp.int32)
    return _sc_gather_add(node_features, total_charge.reshape(N_GRAPHS),
                          W.reshape(D_FEAT), idx)


# speedup vs baseline: 1.1157x; 1.1157x over previous
import functools

import jax
import jax.numpy as jnp
from jax import lax
from jax.experimental import pallas as pl
from jax.experimental.pallas import tpu as pltpu
from jax.experimental.pallas import tpu_sc as plsc

N_NODES = 100000
N_GRAPHS = 256
D_FEAT = 128

NC = 2
NS = 16
NW = NC * NS
LANES = 16
NJ = D_FEAT // LANES

CHUNK = 160
NCHUNKS = N_NODES // CHUNK
NSLOT = 4
T_MAX = (NCHUNKS + NW - 1) // NW

_MESH = plsc.VectorSubcoreMesh(core_axis_name="c", subcore_axis_name="s")


@functools.partial(
    pl.kernel,
    out_type=jax.ShapeDtypeStruct((N_NODES, D_FEAT), jnp.float32),
    mesh=_MESH,
    scratch_types=[
        pltpu.VMEM((N_GRAPHS, D_FEAT), jnp.float32),
        pltpu.VMEM((N_GRAPHS,), jnp.float32),
        pltpu.VMEM((D_FEAT,), jnp.float32),
        pltpu.VMEM((CHUNK, D_FEAT), jnp.float32),
        pltpu.VMEM((CHUNK, D_FEAT), jnp.float32),
        pltpu.VMEM((CHUNK, D_FEAT), jnp.float32),
        pltpu.VMEM((CHUNK, D_FEAT), jnp.float32),
        pltpu.VMEM((CHUNK,), jnp.int32),
        pltpu.VMEM((CHUNK,), jnp.int32),
        pltpu.VMEM((CHUNK,), jnp.int32),
        pltpu.VMEM((CHUNK,), jnp.int32),
        pltpu.SemaphoreType.DMA,
        pltpu.SemaphoreType.DMA,
        pltpu.SemaphoreType.DMA,
        pltpu.SemaphoreType.DMA,
        pltpu.SemaphoreType.DMA,
        pltpu.SemaphoreType.DMA,
        pltpu.SemaphoreType.DMA,
        pltpu.SemaphoreType.DMA,
    ],
)
def _sc_gather_add(nf_hbm, tc_hbm, w_hbm, idx_hbm, out_hbm,
                   emb_v, tc_v, w_v, nf0, nf1, nf2, nf3, ix0, ix1, ix2, ix3,
                   in0, in1, in2, in3, out0, out1, out2, out3):
    nf_bufs = (nf0, nf1, nf2, nf3)
    idx_bufs = (ix0, ix1, ix2, ix3)
    in_sems = (in0, in1, in2, in3)
    out_sems = (out0, out1, out2, out3)
    wid = lax.axis_index("s") * NC + lax.axis_index("c")

    def chunk_of(t):
        return wid + t * NW

    def fire_in(t, b):
        base = chunk_of(t) * CHUNK
        pltpu.async_copy(idx_hbm.at[pl.ds(base, CHUNK)], idx_bufs[b],
                         in_sems[b])
        pltpu.async_copy(nf_hbm.at[pl.ds(base, CHUNK)], nf_bufs[b],
                         in_sems[b])

    def wait_in(t, b):
        base = chunk_of(t) * CHUNK
        pltpu.make_async_copy(idx_hbm.at[pl.ds(base, CHUNK)], idx_bufs[b],
                              in_sems[b]).wait()
        pltpu.make_async_copy(nf_hbm.at[pl.ds(base, CHUNK)], nf_bufs[b],
                              in_sems[b]).wait()

    def fire_out(t, b):
        base = chunk_of(t) * CHUNK
        pltpu.async_copy(nf_bufs[b], out_hbm.at[pl.ds(base, CHUNK)],
                         out_sems[b])

    def wait_out(t, b):
        base = chunk_of(t) * CHUNK
        pltpu.make_async_copy(nf_bufs[b], out_hbm.at[pl.ds(base, CHUNK)],
                              out_sems[b]).wait()

    def compute(b):
        nf = nf_bufs[b]
        ix = idx_bufs[b]
        first = ix[pl.ds(0, LANES)][0]
        last = ix[pl.ds(CHUNK - LANES, LANES)][LANES - 1]

        @pl.when(first == last)
        def _uniform():
            rows = [emb_v[first, pl.ds(j * LANES, LANES)] for j in range(NJ)]

            @pl.loop(0, CHUNK, step=LANES)
            def _grp(i0):
                for lane in range(LANES):
                    for j in range(NJ):
                        sl = pl.ds(j * LANES, LANES)
                        plsc.addupdate(nf.at[i0 + lane, sl], rows[j])

        @pl.when(first != last)
        def _mixed():
            @pl.loop(0, CHUNK, step=LANES)
            def _grp(i0):
                bvec = ix[pl.ds(i0, LANES)]
                gf = bvec[0]
                gl = bvec[LANES - 1]

                @pl.when(gf == gl)
                def _grp_uniform():
                    grows = [emb_v[gf, pl.ds(j * LANES, LANES)]
                             for j in range(NJ)]
                    for lane in range(LANES):
                        for j in range(NJ):
                            sl = pl.ds(j * LANES, LANES)
                            plsc.addupdate(nf.at[i0 + lane, sl], grows[j])

                @pl.when(gf != gl)
                def _grp_mixed():
                    for lane in range(LANES):
                        g = bvec[lane]
                        for j in range(NJ):
                            sl = pl.ds(j * LANES, LANES)
                            plsc.addupdate(nf.at[i0 + lane, sl], emb_v[g, sl])

    for t0 in range(2):
        @pl.when(chunk_of(t0) < NCHUNKS)
        def _(t0=t0):
            fire_in(t0, t0 % NSLOT)

    pltpu.sync_copy(tc_hbm, tc_v)
    pltpu.sync_copy(w_hbm, w_v)
    wrow = [w_v[pl.ds(j * LANES, LANES)] for j in range(NJ)]

    @pl.loop(0, N_GRAPHS, step=LANES)
    def _build(g0):
        tvec = tc_v[pl.ds(g0, LANES)]
        for lane in range(LANES):
            s = tvec[lane]
            for j in range(NJ):
                emb_v[g0 + lane, pl.ds(j * LANES, LANES)] = wrow[j] * s

    @pl.loop(0, T_MAX, step=NSLOT)
    def _body(tt):
        for k in range(NSLOT):
            t = tt + k
            ci = chunk_of(t)

            @pl.when(ci < NCHUNKS)
            def _(t=t, b=k, ci=ci):
                wait_in(t, b)
                compute(b)
                fire_out(t, b)
                b2 = (b + 2) % NSLOT

                @pl.when(jnp.logical_and(t >= 2, chunk_of(t + 2) < NCHUNKS))
                def _():
                    wait_out(t - 2, b2)

                @pl.when(chunk_of(t + 2) < NCHUNKS)
                def _():
                    fire_in(t + 2, b2)

    nchunks_mine = (NCHUNKS - wid + NW - 1) // NW

    for k in range(NSLOT):
        t_last = nchunks_mine - 1 - k

        @pl.when(t_last >= 0)
        def _(t_last=t_last):
            for b in range(NSLOT):
                @pl.when(t_last % NSLOT == b)
                def _(b=b):
                    wait_out(t_last, b)


def kernel(node_features, total_charge, batch, W):
    idx = batch.astype(jnp.int32)
    return _sc_gather_add(node_features, total_charge.reshape(N_GRAPHS),
                          W.reshape(D_FEAT), idx)
